# baseline (speedup 1.0000x reference)
import functools

import jax
import jax.numpy as jnp
from jax import lax
from jax.experimental import pallas as pl
from jax.experimental.pallas import tpu as pltpu
from jax.experimental.pallas import tpu_sc as plsc


def kernel(x, pos_weight):
    B, L, H = x.shape
    NW = 32
    rows_per_w = L // NW
    R = 16
    n_chunks = rows_per_w // R
    n_col = H // 16

    mesh = plsc.VectorSubcoreMesh(core_axis_name="c", subcore_axis_name="s")

    @functools.partial(
        pl.kernel,
        mesh=mesh,
        out_type=jax.ShapeDtypeStruct((B, L, H), jnp.float32),
        scratch_types=[
            pltpu.VMEM((R, H), jnp.float32),
            pltpu.VMEM((R, H), jnp.float32),
            pltpu.VMEM((R, H), jnp.float32),
            pltpu.VMEM((R, H), jnp.float32),
            pltpu.VMEM((R, H), jnp.float32),
            pltpu.VMEM((R, H), jnp.float32),
            pltpu.SemaphoreType.DMA,
            pltpu.SemaphoreType.DMA,
            pltpu.SemaphoreType.DMA,
            pltpu.SemaphoreType.DMA,
            pltpu.SemaphoreType.DMA,
            pltpu.SemaphoreType.DMA,
            pltpu.SemaphoreType.DMA,
            pltpu.SemaphoreType.DMA,
            pltpu.SemaphoreType.DMA,
            pltpu.SemaphoreType.DMA,
        ],
    )
    def k(x_hbm, w_hbm, o_hbm, xv0, xv1, xv2, xv3, wv0, wv1,
          si0, si1, si2, si3, so0, so1, so2, so3, sw0, sw1):
        cid = lax.axis_index("c")
        sid = lax.axis_index("s")
        wid = sid * 2 + cid
        base = wid * rows_per_w

        xvs = [xv0, xv1, xv2, xv3]
        wvs = [wv0, wv1]
        sins, souts, sws = [si0, si1, si2, si3], [so0, so1, so2, so3], [sw0, sw1]

        def w_copy(t, par):
            return pltpu.make_async_copy(
                w_hbm.at[pl.ds(base + t * R, R)], wvs[par], sws[par])

        def in_copy(t, b):
            return pltpu.make_async_copy(
                x_hbm.at[b, pl.ds(base + t * R, R)], xvs[b], sins[b])

        def out_copy(t, b):
            return pltpu.make_async_copy(
                xvs[b], o_hbm.at[b, pl.ds(base + t * R, R)], souts[b])

        def step(t, b, par, first=False, last=False):
            in_copy(t, b).wait()
            if b == 0:
                w_copy(t, par).wait()
            if b < 2:
                if not first:
                    out_copy(t - 1, b + 2).wait()
                in_copy(t, b + 2).start()
            else:
                out_copy(t, b - 2).wait()
                if not last:
                    in_copy(t + 1, b - 2).start()

            xv, wv = xvs[b], wvs[par]

            @plsc.parallel_loop(0, R, 1, unroll=2)
            def add_body(r, xv=xv, wv=wv):
                for c in range(n_col):
                    sl = pl.ds(c * 16, 16)
                    xv[r, sl] = xv[r, sl] + wv[r, sl]

            out_copy(t, b).start()
            if b == B - 1 and not last:
                @pl.when(t + 2 < n_chunks)
                def _():
                    w_copy(t + 2, par).start()

        w_copy(0, 0).start()
        w_copy(1, 1).start()
        in_copy(0, 0).start()
        in_copy(0, 1).start()

        for b in range(B):
            step(0, b, 0, first=True)

        def mid_body(t2, carry):
            for b in range(B):
                step(2 * t2 + 1, b, 1)
            for b in range(B):
                step(2 * t2 + 2, b, 0)
            return carry

        lax.fori_loop(0, (n_chunks - 2) // 2, mid_body, 0)

        for b in range(B):
            step(n_chunks - 1, b, 1, last=True)

        out_copy(n_chunks - 1, 2).wait()
        out_copy(n_chunks - 1, 3).wait()

    return k(x, pos_weight)


# speedup vs baseline: 1.0072x; 1.0072x over previous
import functools

import jax
import jax.numpy as jnp
from jax import lax
from jax.experimental import pallas as pl
from jax.experimental.pallas import tpu as pltpu
from jax.experimental.pallas import tpu_sc as plsc


def kernel(x, pos_weight):
    B, L, H = x.shape
    NW = 32
    rows_per_w = L // NW
    R = 32
    n_chunks = rows_per_w // R
    n_col = H // 16

    mesh = plsc.VectorSubcoreMesh(core_axis_name="c", subcore_axis_name="s")

    @functools.partial(
        pl.kernel,
        mesh=mesh,
        out_type=jax.ShapeDtypeStruct((B, L, H), jnp.float32),
        scratch_types=[
            pltpu.VMEM((R, H), jnp.float32),
            pltpu.VMEM((R, H), jnp.float32),
            pltpu.VMEM((R, H), jnp.float32),
            pltpu.VMEM((R, H), jnp.float32),
            pltpu.VMEM((R, H), jnp.float32),
            pltpu.SemaphoreType.DMA,
            pltpu.SemaphoreType.DMA,
            pltpu.SemaphoreType.DMA,
            pltpu.SemaphoreType.DMA,
            pltpu.SemaphoreType.DMA,
            pltpu.SemaphoreType.DMA,
            pltpu.SemaphoreType.DMA,
            pltpu.SemaphoreType.DMA,
            pltpu.SemaphoreType.DMA,
        ],
    )
    def k(x_hbm, w_hbm, o_hbm, xv0, xv1, xv2, xv3, wv,
          si0, si1, si2, si3, so0, so1, so2, so3, sw):
        cid = lax.axis_index("c")
        sid = lax.axis_index("s")
        wid = sid * 2 + cid
        base = wid * rows_per_w

        xvs = [xv0, xv1, xv2, xv3]
        sins, souts = [si0, si1, si2, si3], [so0, so1, so2, so3]

        def w_copy(t):
            return pltpu.make_async_copy(
                w_hbm.at[pl.ds(base + t * R, R)], wv, sw)

        def in_copy(t, b):
            return pltpu.make_async_copy(
                x_hbm.at[b, pl.ds(base + t * R, R)], xvs[b], sins[b])

        def out_copy(t, b):
            return pltpu.make_async_copy(
                xvs[b], o_hbm.at[b, pl.ds(base + t * R, R)], souts[b])

        def step(t, b, first=False, last=False):
            in_copy(t, b).wait()
            if b == 0:
                w_copy(t).wait()
            if b < 2:
                if not first:
                    out_copy(t - 1, b + 2).wait()
                in_copy(t, b + 2).start()
            else:
                out_copy(t, b - 2).wait()
                if not last:
                    in_copy(t + 1, b - 2).start()

            xv = xvs[b]

            @plsc.parallel_loop(0, R, 1, unroll=2)
            def add_body(r, xv=xv):
                for c in range(n_col):
                    sl = pl.ds(c * 16, 16)
                    xv[r, sl] = xv[r, sl] + wv[r, sl]

            out_copy(t, b).start()
            if b == B - 1 and not last:
                w_copy(t + 1).start()

        w_copy(0).start()
        in_copy(0, 0).start()
        in_copy(0, 1).start()

        for b in range(B):
            step(0, b, first=True)

        def mid_body(t, carry):
            for b in range(B):
                step(t, b)
            return carry

        lax.fori_loop(1, n_chunks - 1, mid_body, 0)

        for b in range(B):
            step(n_chunks - 1, b, last=True)

        out_copy(n_chunks - 1, 2).wait()
        out_copy(n_chunks - 1, 3).wait()

    return k(x, pos_weight)
